# Initial kernel scaffold; baseline (speedup 1.0000x reference)
import jax
import jax.numpy as jnp
from jax.experimental import pallas as pl


def kernel(rays_o, rays_d, rays_d_norm, rays_near, rays_far, t_nears, t_fars, table, beta, ray_indices):
    raise NotImplementedError("write your pallas kernel here")


# speedup vs baseline: 3.7675x; 3.7675x over previous
import functools

import jax
import jax.numpy as jnp
from jax import lax
from jax.experimental import pallas as pl
from jax.experimental.pallas import tpu as pltpu
from jax.experimental.pallas import tpu_sc as plsc

_VOX = 0.015
_INV_VOX = 1.0 / _VOX
_NEMB = 200000
_NRAYS = 4096
_NSAMP = 131072
_NC = 2
_NS = 16
_NW = _NC * _NS
_CHUNK = _NSAMP // _NW
_NGRP = _CHUNK // 16
_H1 = 2654435761
_H2 = 805459861
_NEG = -3.0e38

_f32 = jnp.float32
_i32 = jnp.int32


def _floor_frac(x):
    xi = x * _f32(_INV_VOX)
    t = xi.astype(_i32)
    tf = t.astype(_f32)
    b = t - (tf > xi).astype(_i32)
    f = xi - b.astype(_f32)
    return b, f


def _hash_parts(b, mult):
    u0 = plsc.bitcast(b, jnp.uint32) * jnp.uint32(mult)
    u1 = plsc.bitcast(b + 1, jnp.uint32) * jnp.uint32(mult)
    return u0, u1


def _sh_comps(dx, dy, dz):
    one = jnp.full((16,), 1.0, _f32)
    return [
        _f32(0.28209479177387814) * one,
        _f32(0.4886025119029199) * dy,
        _f32(0.4886025119029199) * dz,
        _f32(0.4886025119029199) * dx,
        _f32(1.0925484305920792) * dx * dy,
        _f32(1.0925484305920792) * dy * dz,
        _f32(0.9461746957575601) * dz * dz - _f32(0.31539156525252),
        _f32(1.0925484305920792) * dx * dz,
        _f32(0.5462742152960396) * (dx * dx - dy * dy),
    ]


def _k1_body(ro_hbm, rd_hbm, tn_hbm, tf_hbm, rid_hbm, table_hbm, beta_hbm,
             pert_hbm, tails_hbm, has_hbm,
             ro_v, rd_v, tn_v, tf_v, rid_v, beta_v,
             idx0, idx1, rows0, rows1,
             s_r0, s_r1, s_r2, s_tm, s_sd, s_gx, s_gy, s_gz,
             sem0, sem1):
    cid = lax.axis_index("c")
    sid = lax.axis_index("s")
    wid = sid * _NC + cid
    base = wid * _CHUNK
    lane = lax.iota(_i32, 16)

    pltpu.sync_copy(ro_hbm, ro_v)
    pltpu.sync_copy(rd_hbm, rd_v)
    pltpu.sync_copy(beta_hbm, beta_v)
    pltpu.sync_copy(tn_hbm.at[pl.ds(base, _CHUNK)], tn_v)
    pltpu.sync_copy(tf_hbm.at[pl.ds(base, _CHUNK)], tf_v)

    @pl.when(wid == 0)
    def _():
        rid_v[pl.ds(0, 16)] = jnp.full((16,), -1, _i32)
        pltpu.sync_copy(rid_hbm.at[pl.ds(0, _CHUNK)], rid_v.at[pl.ds(16, _CHUNK)])

    @pl.when(wid > 0)
    def _():
        pltpu.sync_copy(rid_hbm.at[pl.ds(base - 16, _CHUNK + 16)], rid_v)

    bvec = _f32(_VOX) + jnp.abs(beta_v[...])
    inv_b = 1.0 / bvec

    def pregather(g, idx_ref):
        off = g * 16
        rid_g = rid_v[pl.ds(16 + off, 16)]
        tn_g = tn_v[pl.ds(off, 16)]
        tf_g = tf_v[pl.ds(off, 16)]
        tmid = _f32(0.5) * (tn_g + tf_g)
        dt = tf_g - tn_g
        b3 = rid_g * 3
        ox = plsc.load_gather(ro_v, [b3])
        oy = plsc.load_gather(ro_v, [b3 + 1])
        oz = plsc.load_gather(ro_v, [b3 + 2])
        ddx = plsc.load_gather(rd_v, [b3])
        ddy = plsc.load_gather(rd_v, [b3 + 1])
        ddz = plsc.load_gather(rd_v, [b3 + 2])
        b0, f0 = _floor_frac(ox + tmid * ddx)
        b1, f1 = _floor_frac(oy + tmid * ddy)
        b2, f2 = _floor_frac(oz + tmid * ddz)
        hx = _hash_parts(b0, 1)
        hy = _hash_parts(b1, _H1)
        hz = _hash_parts(b2, _H2)
        for c in range(8):
            dxb, dyb, dzb = (c >> 2) & 1, (c >> 1) & 1, c & 1
            h = (hx[dxb] ^ hy[dyb] ^ hz[dzb]) % jnp.uint32(_NEMB)
            idx_ref[pl.ds(c * 16, 16)] = h.astype(_i32)
        wx = [1.0 - f0, f0]
        wy = [1.0 - f1, f1]
        wz = [1.0 - f2, f2]
        wyz = [wy[j >> 1] * wz[j & 1] for j in range(4)]
        wxz = [wx[j >> 1] * wz[j & 1] for j in range(4)]
        wxy = [wx[j >> 1] * wy[j & 1] for j in range(4)]
        w8 = [wx[(c >> 2) & 1] * wyz[c & 3] for c in range(8)]
        comps = _sh_comps(ddx, ddy, ddz)
        return (rid_g, tmid, dt, w8, wyz, wxz, wxy, comps)

    def combine(g, rows_ref, pre, carry):
        acc, hasc = carry
        rid_g, tmid, dt, w8, wyz, wxz, wxy, comps = pre
        off = g * 16
        r0 = []
        col0 = jnp.full((16,), 0, _i32)
        for c in range(8):
            rowv = lane + (c * 16)
            r0.append(plsc.load_gather(rows_ref, [rowv, col0]))
        sdfs = jnp.zeros((16,), _f32)
        for c in range(8):
            sdfs = sdfs + w8[c] * r0[c]
        gx = jnp.zeros((16,), _f32)
        gy = jnp.zeros((16,), _f32)
        gz = jnp.zeros((16,), _f32)
        for j in range(4):
            gx = gx + wyz[j] * (r0[4 + j] - r0[j])
        for j in range(4):
            dxb, dzb = j >> 1, j & 1
            gy = gy + wxz[j] * (r0[dxb * 4 + 2 + dzb] - r0[dxb * 4 + dzb])
        for j in range(4):
            gz = gz + wxy[j] * (r0[2 * j + 1] - r0[2 * j])
        gx = gx * _f32(_INV_VOX)
        gy = gy * _f32(_INV_VOX)
        gz = gz * _f32(_INV_VOX)
        logits = [jnp.zeros((16,), _f32) for _ in range(3)]
        for k in range(3):
            for j in range(9):
                ch = 1 + k * 9 + j
                colv = jnp.full((16,), ch, _i32)
                e = jnp.zeros((16,), _f32)
                for c in range(8):
                    rowv = lane + (c * 16)
                    e = e + w8[c] * plsc.load_gather(rows_ref, [rowv, colv])
                logits[k] = logits[k] + e * comps[j]
        rgb = [1.0 / (1.0 + jnp.exp(-logits[k])) for k in range(3)]
        sgn = jnp.sign(sdfs)
        em1 = jnp.exp(-jnp.abs(sdfs) * inv_b) - 1.0
        sig = inv_b * (_f32(0.5) + _f32(0.5) * sgn * em1)
        sd_g = sig * dt
        s_r0[pl.ds(off, 16)] = rgb[0]
        s_r1[pl.ds(off, 16)] = rgb[1]
        s_r2[pl.ds(off, 16)] = rgb[2]
        s_tm[pl.ds(off, 16)] = tmid
        s_sd[pl.ds(off, 16)] = sd_g
        s_gx[pl.ds(off, 16)] = gx
        s_gy[pl.ds(off, 16)] = gy
        s_gz[pl.ds(off, 16)] = gz
        prev_g = plsc.load_gather(rid_v, [lane + (15 + off)])
        flag = rid_g != prev_g
        cin = plsc.cumsum(sd_g)
        excl = cin - sd_g
        total = jnp.max(cin)
        sv = jnp.where(flag, excl, _f32(_NEG))
        m = jnp.max(sv)
        hasg = jnp.max(flag.astype(_f32))
        acc = jnp.where(hasg > 0, total - m, acc + total)
        hasc = jnp.maximum(hasc, hasg)
        return (acc, hasc)

    def body(gg, carry):
        g0 = gg * 2
        g1 = g0 + 1
        pre0 = pregather(g0, idx0)
        cp0 = pltpu.async_copy(table_hbm.at[idx0], rows0, sem0)
        pre1 = pregather(g1, idx1)
        cp1 = pltpu.async_copy(table_hbm.at[idx1], rows1, sem1)
        cp0.wait()
        carry = combine(g0, rows0, pre0, carry)
        cp1.wait()
        carry = combine(g1, rows1, pre1, carry)
        return carry

    acc, hasc = lax.fori_loop(0, _NGRP // 2, body, (_f32(0.0), _f32(0.0)))

    for r, ref in enumerate((s_r0, s_r1, s_r2, s_tm, s_sd, s_gx, s_gy, s_gz)):
        pltpu.sync_copy(ref, pert_hbm.at[r, pl.ds(base, _CHUNK)])
    tn_v[pl.ds(0, 16)] = jnp.broadcast_to(acc, (16,))
    pltpu.sync_copy(tn_v.at[pl.ds(0, 16)], tails_hbm.at[wid])
    tf_v[pl.ds(0, 16)] = jnp.broadcast_to(hasc, (16,))
    pltpu.sync_copy(tf_v.at[pl.ds(0, 16)], has_hbm.at[wid])


def _k3_body(pert_hbm, rid_hbm, tails_hbm, has_hbm, zeros_hbm,
             part_hbm, grads_hbm,
             s_r0, s_r1, s_r2, s_tm, s_sd, s_gx, s_gy, s_gz,
             rid_v, rid2d, tails_v, has_v, scr, vals_v, grads_v,
             acc_sh):
    cid = lax.axis_index("c")
    sid = lax.axis_index("s")
    wid = sid * _NC + cid
    base = wid * _CHUNK
    lane = lax.iota(_i32, 16)

    pltpu.sync_copy(zeros_hbm, vals_v.at[pl.ds(0, 256)])
    pltpu.sync_copy(vals_v.at[pl.ds(0, 256)], acc_sh.at[pl.ds(sid * 256, 256)])

    for r, ref in enumerate((s_r0, s_r1, s_r2, s_tm, s_sd, s_gx, s_gy, s_gz)):
        pltpu.sync_copy(pert_hbm.at[r, pl.ds(base, _CHUNK)], ref)

    @pl.when(wid == 0)
    def _():
        rid_v[pl.ds(0, 16)] = jnp.full((16,), -1, _i32)
        pltpu.sync_copy(rid_hbm.at[pl.ds(0, _CHUNK)], rid_v.at[pl.ds(16, _CHUNK)])

    @pl.when(wid > 0)
    def _():
        pltpu.sync_copy(rid_hbm.at[pl.ds(base - 16, _CHUNK + 16)], rid_v)

    for j in range(32):
        pltpu.sync_copy(rid_hbm.at[pl.ds(base + j * 128, 128)], rid2d.at[j])

    pltpu.sync_copy(tails_hbm, tails_v)
    pltpu.sync_copy(has_hbm, has_v)

    z16 = jnp.full((16,), 0, _i32)
    ta = plsc.load_gather(tails_v, [lane, z16])
    tb = plsc.load_gather(tails_v, [lane + 16, z16])
    ha = plsc.load_gather(has_v, [lane, z16])
    hb = plsc.load_gather(has_v, [lane + 16, z16])
    pia = plsc.cumsum(ta)
    pib = plsc.cumsum(tb) + jnp.max(pia)
    pea = pia - ta
    peb = pib - tb
    sva = jnp.where(ha > 0, pea, _f32(_NEG))
    svb = jnp.where(hb > 0, peb, _f32(_NEG))
    cma = plsc.cummax(sva)
    cmb = jnp.maximum(plsc.cummax(svb), jnp.max(cma))
    scr[pl.ds(0, 16)] = jnp.broadcast_to(_f32(0.0), (16,))
    scr[pl.ds(16, 16)] = cma
    scr[pl.ds(32, 16)] = cmb
    e2a = plsc.load_gather(scr, [lane + 15])
    e2b = plsc.load_gather(scr, [lane + 31])
    cva = pea - e2a
    cvb = peb - e2b
    sel_a = jnp.sum(jnp.where(lane == wid, cva, _f32(0.0)))
    sel_b = jnp.sum(jnp.where(lane == (wid - 16), cvb, _f32(0.0)))
    carry0 = jnp.where(wid < 16, sel_a, sel_b)

    plsc.subcore_barrier()

    def body(g, acc):
        off = g * 16
        sd_g = s_sd[pl.ds(off, 16)]
        rid_g = rid_v[pl.ds(16 + off, 16)]
        prev_g = plsc.load_gather(rid_v, [lane + (15 + off)])
        flag = rid_g != prev_g
        cin = plsc.cumsum(sd_g)
        excl = cin - sd_g
        total = jnp.max(cin)
        sv = jnp.where(flag, excl, _f32(_NEG))
        cmx = plsc.cummax(sv)
        e_lane = jnp.maximum(cmx, -acc)
        slocal = excl - e_lane
        trans = jnp.exp(-slocal)
        alpha = 1.0 - jnp.exp(-sd_g)
        wgt = alpha * trans
        m = jnp.max(sv)
        hasg = jnp.max(flag.astype(_f32))
        acc = jnp.where(hasg > 0, total - m, acc + total)
        gx = s_gx[pl.ds(off, 16)]
        gy = s_gy[pl.ds(off, 16)]
        gz = s_gz[pl.ds(off, 16)]
        gg = gx * gx + gy * gy + gz * gz
        bits = plsc.bitcast(gg, _i32)
        bits = jnp.full((16,), 0x5F3759DF, _i32) - lax.shift_right_logical(bits, 1)
        y = plsc.bitcast(bits, _f32)
        for _ in range(3):
            y = y * (_f32(1.5) - _f32(0.5) * gg * y * y)
        nlen = gg * y
        inv = 1.0 / jnp.maximum(nlen, _f32(1e-12))
        rowi = lane + off
        for k, gv in enumerate((gx, gy, gz)):
            plsc.store_scatter(grads_v, [rowi, jnp.full((16,), k, _i32)], gv)
        vals = (wgt * s_r0[pl.ds(off, 16)],
                wgt * s_r1[pl.ds(off, 16)],
                wgt * s_r2[pl.ds(off, 16)],
                wgt * s_tm[pl.ds(off, 16)],
                wgt * (gx * inv),
                wgt * (gy * inv),
                wgt * (gz * inv),
                wgt)
        for ch, v in enumerate(vals):
            plsc.store_scatter(vals_v, [rowi, jnp.full((16,), ch, _i32)], v)
        return acc

    lax.fori_loop(0, _NGRP, body, carry0)

    pltpu.sync_copy(grads_v, grads_hbm.at[pl.ds(base, _CHUNK)])

    def scat(j, carry):
        pltpu.sync_copy(vals_v.at[pl.ds(j * 128, 128)], acc_sh.at[rid2d.at[j]],
                        add=True)
        return carry

    lax.fori_loop(0, 32, scat, 0)

    plsc.subcore_barrier()

    @pl.when(sid == 0)
    def _():
        pltpu.sync_copy(acc_sh, part_hbm.at[cid])


def _k4_body(p_ref, rn_ref, rf_ref, rdn_ref,
             rgb_ref, depth_ref, nrm_ref, acc_ref, near_ref, far_ref):
    s = p_ref[0] + p_ref[1]
    rdn = rdn_ref[...]
    rgb_ref[...] = s[:, 0:3]
    depth_ref[...] = s[:, 3:4] / rdn
    nrm_ref[...] = s[:, 4:7]
    acc_ref[...] = s[:, 7:8]
    near_ref[...] = rn_ref[...] / rdn
    far_ref[...] = rf_ref[...] / rdn


_mesh = plsc.VectorSubcoreMesh(core_axis_name="c", subcore_axis_name="s")
_sc_params = pltpu.CompilerParams(needs_layout_passes=False,
                                  use_tc_tiling_on_sc=False)

_k1 = functools.partial(
    pl.kernel,
    out_type=(
        jax.ShapeDtypeStruct((8, _NSAMP), _f32),
        jax.ShapeDtypeStruct((_NW, 16), _f32),
        jax.ShapeDtypeStruct((_NW, 16), _f32),
    ),
    mesh=_mesh,
    compiler_params=_sc_params,
    scratch_types=[
        pltpu.VMEM((_NRAYS * 3,), _f32),
        pltpu.VMEM((_NRAYS * 3,), _f32),
        pltpu.VMEM((_CHUNK,), _f32),
        pltpu.VMEM((_CHUNK,), _f32),
        pltpu.VMEM((_CHUNK + 16,), _i32),
        pltpu.VMEM((16,), _f32),
        pltpu.VMEM((128,), _i32),
        pltpu.VMEM((128,), _i32),
        pltpu.VMEM((128, 32), _f32),
        pltpu.VMEM((128, 32), _f32),
    ] + [pltpu.VMEM((_CHUNK,), _f32)] * 8
    + [pltpu.SemaphoreType.DMA, pltpu.SemaphoreType.DMA],
)(_k1_body)

_k3 = functools.partial(
    pl.kernel,
    out_type=(
        jax.ShapeDtypeStruct((_NC, _NRAYS, 8), _f32),
        jax.ShapeDtypeStruct((_NSAMP, 3), _f32),
    ),
    mesh=_mesh,
    compiler_params=_sc_params,
    scratch_types=[pltpu.VMEM((_CHUNK,), _f32)] * 8
    + [
        pltpu.VMEM((_CHUNK + 16,), _i32),
        pltpu.VMEM((32, 128), _i32),
        pltpu.VMEM((_NW, 16), _f32),
        pltpu.VMEM((_NW, 16), _f32),
        pltpu.VMEM((48,), _f32),
        pltpu.VMEM((_CHUNK, 8), _f32),
        pltpu.VMEM((_CHUNK, 3), _f32),
        pltpu.VMEM_SHARED((_NRAYS, 8), _f32),
    ],
)(_k3_body)


def kernel(rays_o, rays_d, rays_d_norm, rays_near, rays_far, t_nears, t_fars,
           table, beta, ray_indices):
    tablep = jnp.pad(table, ((0, 0), (0, 4)))
    ro = rays_o.reshape(-1)
    rd = rays_d.reshape(-1)
    tn = t_nears.reshape(-1)
    tf = t_fars.reshape(-1)
    rid = ray_indices.astype(_i32)
    beta16 = jnp.broadcast_to(beta.reshape(1), (16,))
    zeros256 = jnp.zeros((256, 8), _f32)

    pert, tails, has = _k1(ro, rd, tn, tf, rid, tablep, beta16)
    partials, grads = _k3(pert, rid, tails, has, zeros256)

    rgb, depth, nrm, acc, near, far = pl.pallas_call(
        _k4_body,
        out_shape=(
            jax.ShapeDtypeStruct((_NRAYS, 3), _f32),
            jax.ShapeDtypeStruct((_NRAYS, 1), _f32),
            jax.ShapeDtypeStruct((_NRAYS, 3), _f32),
            jax.ShapeDtypeStruct((_NRAYS, 1), _f32),
            jax.ShapeDtypeStruct((_NRAYS, 1), _f32),
            jax.ShapeDtypeStruct((_NRAYS, 1), _f32),
        ),
    )(partials, rays_near, rays_far, rays_d_norm)

    return (rgb, depth, nrm, acc, grads, near, far)


# speedup vs baseline: 4.1596x; 1.1041x over previous
import functools

import jax
import jax.numpy as jnp
from jax import lax
from jax.experimental import pallas as pl
from jax.experimental.pallas import tpu as pltpu
from jax.experimental.pallas import tpu_sc as plsc

_VOX = 0.015
_INV_VOX = 1.0 / _VOX
_NEMB = 200000
_NRAYS = 4096
_NSAMP = 131072
_NC = 2
_NS = 16
_NW = _NC * _NS
_CHUNK = _NSAMP // _NW
_NGRP = _CHUNK // 16
_H1 = 2654435761
_H2 = 805459861
_NEG = -3.0e38

_f32 = jnp.float32
_i32 = jnp.int32


def _floor_frac(x):
    xi = x * _f32(_INV_VOX)
    t = xi.astype(_i32)
    tf = t.astype(_f32)
    b = t - (tf > xi).astype(_i32)
    f = xi - b.astype(_f32)
    return b, f


def _hash_parts(b, mult):
    u0 = plsc.bitcast(b, jnp.uint32) * jnp.uint32(mult)
    u1 = plsc.bitcast(b + 1, jnp.uint32) * jnp.uint32(mult)
    return u0, u1


def _sh_comps(dx, dy, dz):
    one = jnp.full((16,), 1.0, _f32)
    return [
        _f32(0.28209479177387814) * one,
        _f32(0.4886025119029199) * dy,
        _f32(0.4886025119029199) * dz,
        _f32(0.4886025119029199) * dx,
        _f32(1.0925484305920792) * dx * dy,
        _f32(1.0925484305920792) * dy * dz,
        _f32(0.9461746957575601) * dz * dz - _f32(0.31539156525252),
        _f32(1.0925484305920792) * dx * dz,
        _f32(0.5462742152960396) * (dx * dx - dy * dy),
    ]


def _k1_body(ro_hbm, rd_hbm, tn_hbm, tf_hbm, rid_hbm, table_hbm, beta_hbm,
             pert_hbm, tails_hbm, has_hbm,
             ro_v, rd_v, tn_v, tf_v, rid_v, beta_v,
             idx0, idx1, rows0, rows1, sbuf,
             s_r0, s_r1, s_r2, s_tm, s_sd, s_gx, s_gy, s_gz,
             sem0, sem1):
    cid = lax.axis_index("c")
    sid = lax.axis_index("s")
    wid = sid * _NC + cid
    base = wid * _CHUNK
    lane = lax.iota(_i32, 16)

    pltpu.sync_copy(ro_hbm, ro_v)
    pltpu.sync_copy(rd_hbm, rd_v)
    pltpu.sync_copy(beta_hbm, beta_v)
    pltpu.sync_copy(tn_hbm.at[pl.ds(base, _CHUNK)], tn_v)
    pltpu.sync_copy(tf_hbm.at[pl.ds(base, _CHUNK)], tf_v)

    @pl.when(wid == 0)
    def _():
        rid_v[pl.ds(0, 16)] = jnp.full((16,), -1, _i32)
        pltpu.sync_copy(rid_hbm.at[pl.ds(0, _CHUNK)], rid_v.at[pl.ds(16, _CHUNK)])

    @pl.when(wid > 0)
    def _():
        pltpu.sync_copy(rid_hbm.at[pl.ds(base - 16, _CHUNK + 16)], rid_v)

    bvec = _f32(_VOX) + jnp.abs(beta_v[...])
    inv_b = 1.0 / bvec

    def pregather(g, idx_ref):
        off = g * 16
        rid_g = rid_v[pl.ds(16 + off, 16)]
        tn_g = tn_v[pl.ds(off, 16)]
        tf_g = tf_v[pl.ds(off, 16)]
        tmid = _f32(0.5) * (tn_g + tf_g)
        dt = tf_g - tn_g
        b3 = rid_g * 3
        ox = plsc.load_gather(ro_v, [b3])
        oy = plsc.load_gather(ro_v, [b3 + 1])
        oz = plsc.load_gather(ro_v, [b3 + 2])
        ddx = plsc.load_gather(rd_v, [b3])
        ddy = plsc.load_gather(rd_v, [b3 + 1])
        ddz = plsc.load_gather(rd_v, [b3 + 2])
        b0, f0 = _floor_frac(ox + tmid * ddx)
        b1, f1 = _floor_frac(oy + tmid * ddy)
        b2, f2 = _floor_frac(oz + tmid * ddz)
        hx = _hash_parts(b0, 1)
        hy = _hash_parts(b1, _H1)
        hz = _hash_parts(b2, _H2)
        for c in range(8):
            dxb, dyb, dzb = (c >> 2) & 1, (c >> 1) & 1, c & 1
            h = (hx[dxb] ^ hy[dyb] ^ hz[dzb]) % jnp.uint32(_NEMB)
            idx_ref[pl.ds(c * 16, 16)] = h.astype(_i32)
        wx = [1.0 - f0, f0]
        wy = [1.0 - f1, f1]
        wz = [1.0 - f2, f2]
        wyz = [wy[j >> 1] * wz[j & 1] for j in range(4)]
        wxz = [wx[j >> 1] * wz[j & 1] for j in range(4)]
        wxy = [wx[j >> 1] * wy[j & 1] for j in range(4)]
        w8 = [wx[(c >> 2) & 1] * wyz[c & 3] for c in range(8)]
        comps = _sh_comps(ddx, ddy, ddz)
        return (rid_g, tmid, dt, w8, wyz, wxz, wxy, comps)

    def combine(g, rows_ref, pre, carry):
        acc, hasc = carry
        rid_g, tmid, dt, w8, wyz, wxz, wxy, comps = pre
        off = g * 16
        rowvs = [lane + (c * 16) for c in range(8)]
        for d in range(32):
            colv = (lane + d) & 31
            a = w8[0] * plsc.load_gather(rows_ref, [rowvs[0], colv])
            for c in range(1, 8):
                a = a + w8[c] * plsc.load_gather(rows_ref, [rowvs[c], colv])
            plsc.store_scatter(sbuf, [colv * 16 + lane], a)
        emb = [sbuf[pl.ds(ch * 16, 16)] for ch in range(28)]
        sdfs = emb[0]
        col0 = jnp.full((16,), 0, _i32)
        r0 = [plsc.load_gather(rows_ref, [rowvs[c], col0]) for c in range(8)]
        gx = jnp.zeros((16,), _f32)
        gy = jnp.zeros((16,), _f32)
        gz = jnp.zeros((16,), _f32)
        for j in range(4):
            gx = gx + wyz[j] * (r0[4 + j] - r0[j])
        for j in range(4):
            dxb, dzb = j >> 1, j & 1
            gy = gy + wxz[j] * (r0[dxb * 4 + 2 + dzb] - r0[dxb * 4 + dzb])
        for j in range(4):
            gz = gz + wxy[j] * (r0[2 * j + 1] - r0[2 * j])
        gx = gx * _f32(_INV_VOX)
        gy = gy * _f32(_INV_VOX)
        gz = gz * _f32(_INV_VOX)
        logits = [jnp.zeros((16,), _f32) for _ in range(3)]
        for k in range(3):
            for j in range(9):
                logits[k] = logits[k] + emb[1 + k * 9 + j] * comps[j]
        rgb = [1.0 / (1.0 + jnp.exp(-logits[k])) for k in range(3)]
        sgn = jnp.sign(sdfs)
        em1 = jnp.exp(-jnp.abs(sdfs) * inv_b) - 1.0
        sig = inv_b * (_f32(0.5) + _f32(0.5) * sgn * em1)
        sd_g = sig * dt
        s_r0[pl.ds(off, 16)] = rgb[0]
        s_r1[pl.ds(off, 16)] = rgb[1]
        s_r2[pl.ds(off, 16)] = rgb[2]
        s_tm[pl.ds(off, 16)] = tmid
        s_sd[pl.ds(off, 16)] = sd_g
        s_gx[pl.ds(off, 16)] = gx
        s_gy[pl.ds(off, 16)] = gy
        s_gz[pl.ds(off, 16)] = gz
        prev_g = plsc.load_gather(rid_v, [lane + (15 + off)])
        flag = rid_g != prev_g
        cin = plsc.cumsum(sd_g)
        excl = cin - sd_g
        total = jnp.max(cin)
        sv = jnp.where(flag, excl, _f32(_NEG))
        m = jnp.max(sv)
        hasg = jnp.max(flag.astype(_f32))
        acc = jnp.where(hasg > 0, total - m, acc + total)
        hasc = jnp.maximum(hasc, hasg)
        return (acc, hasc)

    def body(gg, carry):
        g0 = gg * 2
        g1 = g0 + 1
        pre0 = pregather(g0, idx0)
        cp0 = pltpu.async_copy(table_hbm.at[idx0], rows0, sem0)
        pre1 = pregather(g1, idx1)
        cp1 = pltpu.async_copy(table_hbm.at[idx1], rows1, sem1)
        cp0.wait()
        carry = combine(g0, rows0, pre0, carry)
        cp1.wait()
        carry = combine(g1, rows1, pre1, carry)
        return carry

    acc, hasc = lax.fori_loop(0, _NGRP // 2, body, (_f32(0.0), _f32(0.0)))

    for r, ref in enumerate((s_r0, s_r1, s_r2, s_tm, s_sd, s_gx, s_gy, s_gz)):
        pltpu.sync_copy(ref, pert_hbm.at[r, pl.ds(base, _CHUNK)])
    tn_v[pl.ds(0, 16)] = jnp.broadcast_to(acc, (16,))
    pltpu.sync_copy(tn_v.at[pl.ds(0, 16)], tails_hbm.at[wid])
    tf_v[pl.ds(0, 16)] = jnp.broadcast_to(hasc, (16,))
    pltpu.sync_copy(tf_v.at[pl.ds(0, 16)], has_hbm.at[wid])


def _k3_body(pert_hbm, rid_hbm, tails_hbm, has_hbm, zeros_hbm,
             part_hbm, grads_hbm,
             s_r0, s_r1, s_r2, s_tm, s_sd, s_gx, s_gy, s_gz,
             rid_v, rid2d, tails_v, has_v, scr, vals_v, grads_v,
             acc_sh):
    cid = lax.axis_index("c")
    sid = lax.axis_index("s")
    wid = sid * _NC + cid
    base = wid * _CHUNK
    lane = lax.iota(_i32, 16)

    pltpu.sync_copy(zeros_hbm, vals_v.at[pl.ds(0, 256)])
    pltpu.sync_copy(vals_v.at[pl.ds(0, 256)], acc_sh.at[pl.ds(sid * 256, 256)])

    for r, ref in enumerate((s_r0, s_r1, s_r2, s_tm, s_sd, s_gx, s_gy, s_gz)):
        pltpu.sync_copy(pert_hbm.at[r, pl.ds(base, _CHUNK)], ref)

    @pl.when(wid == 0)
    def _():
        rid_v[pl.ds(0, 16)] = jnp.full((16,), -1, _i32)
        pltpu.sync_copy(rid_hbm.at[pl.ds(0, _CHUNK)], rid_v.at[pl.ds(16, _CHUNK)])

    @pl.when(wid > 0)
    def _():
        pltpu.sync_copy(rid_hbm.at[pl.ds(base - 16, _CHUNK + 16)], rid_v)

    for j in range(32):
        pltpu.sync_copy(rid_hbm.at[pl.ds(base + j * 128, 128)], rid2d.at[j])

    pltpu.sync_copy(tails_hbm, tails_v)
    pltpu.sync_copy(has_hbm, has_v)

    z16 = jnp.full((16,), 0, _i32)
    ta = plsc.load_gather(tails_v, [lane, z16])
    tb = plsc.load_gather(tails_v, [lane + 16, z16])
    ha = plsc.load_gather(has_v, [lane, z16])
    hb = plsc.load_gather(has_v, [lane + 16, z16])
    pia = plsc.cumsum(ta)
    pib = plsc.cumsum(tb) + jnp.max(pia)
    pea = pia - ta
    peb = pib - tb
    sva = jnp.where(ha > 0, pea, _f32(_NEG))
    svb = jnp.where(hb > 0, peb, _f32(_NEG))
    cma = plsc.cummax(sva)
    cmb = jnp.maximum(plsc.cummax(svb), jnp.max(cma))
    scr[pl.ds(0, 16)] = jnp.broadcast_to(_f32(0.0), (16,))
    scr[pl.ds(16, 16)] = cma
    scr[pl.ds(32, 16)] = cmb
    e2a = plsc.load_gather(scr, [lane + 15])
    e2b = plsc.load_gather(scr, [lane + 31])
    cva = pea - e2a
    cvb = peb - e2b
    sel_a = jnp.sum(jnp.where(lane == wid, cva, _f32(0.0)))
    sel_b = jnp.sum(jnp.where(lane == (wid - 16), cvb, _f32(0.0)))
    carry0 = jnp.where(wid < 16, sel_a, sel_b)

    plsc.subcore_barrier()

    def body(g, acc):
        off = g * 16
        sd_g = s_sd[pl.ds(off, 16)]
        rid_g = rid_v[pl.ds(16 + off, 16)]
        prev_g = plsc.load_gather(rid_v, [lane + (15 + off)])
        flag = rid_g != prev_g
        cin = plsc.cumsum(sd_g)
        excl = cin - sd_g
        total = jnp.max(cin)
        sv = jnp.where(flag, excl, _f32(_NEG))
        cmx = plsc.cummax(sv)
        e_lane = jnp.maximum(cmx, -acc)
        slocal = excl - e_lane
        trans = jnp.exp(-slocal)
        alpha = 1.0 - jnp.exp(-sd_g)
        wgt = alpha * trans
        m = jnp.max(sv)
        hasg = jnp.max(flag.astype(_f32))
        acc = jnp.where(hasg > 0, total - m, acc + total)
        gx = s_gx[pl.ds(off, 16)]
        gy = s_gy[pl.ds(off, 16)]
        gz = s_gz[pl.ds(off, 16)]
        gg = gx * gx + gy * gy + gz * gz
        bits = plsc.bitcast(gg, _i32)
        bits = jnp.full((16,), 0x5F3759DF, _i32) - lax.shift_right_logical(bits, 1)
        y = plsc.bitcast(bits, _f32)
        for _ in range(3):
            y = y * (_f32(1.5) - _f32(0.5) * gg * y * y)
        nlen = gg * y
        inv = 1.0 / jnp.maximum(nlen, _f32(1e-12))
        rowi = lane + off
        for k, gv in enumerate((gx, gy, gz)):
            plsc.store_scatter(grads_v, [rowi, jnp.full((16,), k, _i32)], gv)
        vals = (wgt * s_r0[pl.ds(off, 16)],
                wgt * s_r1[pl.ds(off, 16)],
                wgt * s_r2[pl.ds(off, 16)],
                wgt * s_tm[pl.ds(off, 16)],
                wgt * (gx * inv),
                wgt * (gy * inv),
                wgt * (gz * inv),
                wgt)
        for ch, v in enumerate(vals):
            plsc.store_scatter(vals_v, [rowi, jnp.full((16,), ch, _i32)], v)
        return acc

    lax.fori_loop(0, _NGRP, body, carry0)

    pltpu.sync_copy(grads_v, grads_hbm.at[pl.ds(base, _CHUNK)])

    def scat(j, carry):
        pltpu.sync_copy(vals_v.at[pl.ds(j * 128, 128)], acc_sh.at[rid2d.at[j]],
                        add=True)
        return carry

    lax.fori_loop(0, 32, scat, 0)

    plsc.subcore_barrier()

    @pl.when(sid == 0)
    def _():
        pltpu.sync_copy(acc_sh, part_hbm.at[cid])


def _k4_body(p_ref, rn_ref, rf_ref, rdn_ref,
             rgb_ref, depth_ref, nrm_ref, acc_ref, near_ref, far_ref):
    s = p_ref[0] + p_ref[1]
    rdn = rdn_ref[...]
    rgb_ref[...] = s[:, 0:3]
    depth_ref[...] = s[:, 3:4] / rdn
    nrm_ref[...] = s[:, 4:7]
    acc_ref[...] = s[:, 7:8]
    near_ref[...] = rn_ref[...] / rdn
    far_ref[...] = rf_ref[...] / rdn


_mesh = plsc.VectorSubcoreMesh(core_axis_name="c", subcore_axis_name="s")
_sc_params = pltpu.CompilerParams(needs_layout_passes=False,
                                  use_tc_tiling_on_sc=False)

_k1 = functools.partial(
    pl.kernel,
    out_type=(
        jax.ShapeDtypeStruct((8, _NSAMP), _f32),
        jax.ShapeDtypeStruct((_NW, 16), _f32),
        jax.ShapeDtypeStruct((_NW, 16), _f32),
    ),
    mesh=_mesh,
    compiler_params=_sc_params,
    scratch_types=[
        pltpu.VMEM((_NRAYS * 3,), _f32),
        pltpu.VMEM((_NRAYS * 3,), _f32),
        pltpu.VMEM((_CHUNK,), _f32),
        pltpu.VMEM((_CHUNK,), _f32),
        pltpu.VMEM((_CHUNK + 16,), _i32),
        pltpu.VMEM((16,), _f32),
        pltpu.VMEM((128,), _i32),
        pltpu.VMEM((128,), _i32),
        pltpu.VMEM((128, 32), _f32),
        pltpu.VMEM((128, 32), _f32),
        pltpu.VMEM((512,), _f32),
    ] + [pltpu.VMEM((_CHUNK,), _f32)] * 8
    + [pltpu.SemaphoreType.DMA, pltpu.SemaphoreType.DMA],
)(_k1_body)

_k3 = functools.partial(
    pl.kernel,
    out_type=(
        jax.ShapeDtypeStruct((_NC, _NRAYS, 8), _f32),
        jax.ShapeDtypeStruct((_NSAMP, 3), _f32),
    ),
    mesh=_mesh,
    compiler_params=_sc_params,
    scratch_types=[pltpu.VMEM((_CHUNK,), _f32)] * 8
    + [
        pltpu.VMEM((_CHUNK + 16,), _i32),
        pltpu.VMEM((32, 128), _i32),
        pltpu.VMEM((_NW, 16), _f32),
        pltpu.VMEM((_NW, 16), _f32),
        pltpu.VMEM((48,), _f32),
        pltpu.VMEM((_CHUNK, 8), _f32),
        pltpu.VMEM((_CHUNK, 3), _f32),
        pltpu.VMEM_SHARED((_NRAYS, 8), _f32),
    ],
)(_k3_body)


def kernel(rays_o, rays_d, rays_d_norm, rays_near, rays_far, t_nears, t_fars,
           table, beta, ray_indices):
    tablep = jnp.pad(table, ((0, 0), (0, 4)))
    ro = rays_o.reshape(-1)
    rd = rays_d.reshape(-1)
    tn = t_nears.reshape(-1)
    tf = t_fars.reshape(-1)
    rid = ray_indices.astype(_i32)
    beta16 = jnp.broadcast_to(beta.reshape(1), (16,))
    zeros256 = jnp.zeros((256, 8), _f32)

    pert, tails, has = _k1(ro, rd, tn, tf, rid, tablep, beta16)
    partials, grads = _k3(pert, rid, tails, has, zeros256)

    rgb, depth, nrm, acc, near, far = pl.pallas_call(
        _k4_body,
        out_shape=(
            jax.ShapeDtypeStruct((_NRAYS, 3), _f32),
            jax.ShapeDtypeStruct((_NRAYS, 1), _f32),
            jax.ShapeDtypeStruct((_NRAYS, 3), _f32),
            jax.ShapeDtypeStruct((_NRAYS, 1), _f32),
            jax.ShapeDtypeStruct((_NRAYS, 1), _f32),
            jax.ShapeDtypeStruct((_NRAYS, 1), _f32),
        ),
    )(partials, rays_near, rays_far, rays_d_norm)

    return (rgb, depth, nrm, acc, grads, near, far)


# speedup vs baseline: 4.8299x; 1.1611x over previous
import functools

import jax
import jax.numpy as jnp
from jax import lax
from jax.experimental import pallas as pl
from jax.experimental.pallas import tpu as pltpu
from jax.experimental.pallas import tpu_sc as plsc

_VOX = 0.015
_INV_VOX = 1.0 / _VOX
_NEMB = 200000
_NRAYS = 4096
_NSAMP = 131072
_NC = 2
_NS = 16
_NW = _NC * _NS
_CHUNK = _NSAMP // _NW
_NGRP = _CHUNK // 16
_H1 = 2654435761
_H2 = 805459861
_NEG = -3.0e38

_f32 = jnp.float32
_i32 = jnp.int32


def _floor_frac(x):
    xi = x * _f32(_INV_VOX)
    t = xi.astype(_i32)
    tf = t.astype(_f32)
    b = t - (tf > xi).astype(_i32)
    f = xi - b.astype(_f32)
    return b, f


def _hash_parts(b, mult):
    u0 = plsc.bitcast(b, jnp.uint32) * jnp.uint32(mult)
    u1 = plsc.bitcast(b + 1, jnp.uint32) * jnp.uint32(mult)
    return u0, u1


def _sh_comps(dx, dy, dz):
    one = jnp.full((16,), 1.0, _f32)
    return [
        _f32(0.28209479177387814) * one,
        _f32(0.4886025119029199) * dy,
        _f32(0.4886025119029199) * dz,
        _f32(0.4886025119029199) * dx,
        _f32(1.0925484305920792) * dx * dy,
        _f32(1.0925484305920792) * dy * dz,
        _f32(0.9461746957575601) * dz * dz - _f32(0.31539156525252),
        _f32(1.0925484305920792) * dx * dz,
        _f32(0.5462742152960396) * (dx * dx - dy * dy),
    ]


def _k1_body(ro_hbm, rd_hbm, tn_hbm, tf_hbm, rid_hbm, table_hbm, beta_hbm,
             pert_hbm, tails_hbm, has_hbm,
             ro_v, rd_v, tn_v, tf_v, rid_v, beta_v,
             idx0, idx1, idx2, idx3, rows0, rows1, rows2, rows3, sbuf,
             s_r0, s_r1, s_r2, s_tm, s_sd, s_gx, s_gy, s_gz,
             sem0, sem1, sem2, sem3):
    cid = lax.axis_index("c")
    sid = lax.axis_index("s")
    wid = sid * _NC + cid
    base = wid * _CHUNK
    lane = lax.iota(_i32, 16)

    pltpu.sync_copy(ro_hbm, ro_v)
    pltpu.sync_copy(rd_hbm, rd_v)
    pltpu.sync_copy(beta_hbm, beta_v)
    pltpu.sync_copy(tn_hbm.at[pl.ds(base, _CHUNK)], tn_v)
    pltpu.sync_copy(tf_hbm.at[pl.ds(base, _CHUNK)], tf_v)

    @pl.when(wid == 0)
    def _():
        rid_v[pl.ds(0, 16)] = jnp.full((16,), -1, _i32)
        pltpu.sync_copy(rid_hbm.at[pl.ds(0, _CHUNK)], rid_v.at[pl.ds(16, _CHUNK)])

    @pl.when(wid > 0)
    def _():
        pltpu.sync_copy(rid_hbm.at[pl.ds(base - 16, _CHUNK + 16)], rid_v)

    bvec = _f32(_VOX) + jnp.abs(beta_v[...])
    inv_b = 1.0 / bvec

    def pregather(g, idx_ref):
        off = g * 16
        rid_g = rid_v[pl.ds(16 + off, 16)]
        tn_g = tn_v[pl.ds(off, 16)]
        tf_g = tf_v[pl.ds(off, 16)]
        tmid = _f32(0.5) * (tn_g + tf_g)
        dt = tf_g - tn_g
        b3 = rid_g * 3
        ox = plsc.load_gather(ro_v, [b3])
        oy = plsc.load_gather(ro_v, [b3 + 1])
        oz = plsc.load_gather(ro_v, [b3 + 2])
        ddx = plsc.load_gather(rd_v, [b3])
        ddy = plsc.load_gather(rd_v, [b3 + 1])
        ddz = plsc.load_gather(rd_v, [b3 + 2])
        b0, f0 = _floor_frac(ox + tmid * ddx)
        b1, f1 = _floor_frac(oy + tmid * ddy)
        b2, f2 = _floor_frac(oz + tmid * ddz)
        hx = _hash_parts(b0, 1)
        hy = _hash_parts(b1, _H1)
        hz = _hash_parts(b2, _H2)
        for c in range(8):
            dxb, dyb, dzb = (c >> 2) & 1, (c >> 1) & 1, c & 1
            h = (hx[dxb] ^ hy[dyb] ^ hz[dzb]) % jnp.uint32(_NEMB)
            idx_ref[pl.ds(c * 16, 16)] = h.astype(_i32)
        wx = [1.0 - f0, f0]
        wy = [1.0 - f1, f1]
        wz = [1.0 - f2, f2]
        wyz = [wy[j >> 1] * wz[j & 1] for j in range(4)]
        wxz = [wx[j >> 1] * wz[j & 1] for j in range(4)]
        wxy = [wx[j >> 1] * wy[j & 1] for j in range(4)]
        w8 = [wx[(c >> 2) & 1] * wyz[c & 3] for c in range(8)]
        comps = _sh_comps(ddx, ddy, ddz)
        return (rid_g, tmid, dt, w8, wyz, wxz, wxy, comps)

    def combine(g, rows_ref, pre, carry):
        acc, hasc = carry
        rid_g, tmid, dt, w8, wyz, wxz, wxy, comps = pre
        off = g * 16
        rowvs = [lane + (c * 16) for c in range(8)]
        for d in range(32):
            colv = (lane + d) & 31
            a = w8[0] * plsc.load_gather(rows_ref, [rowvs[0], colv])
            for c in range(1, 8):
                a = a + w8[c] * plsc.load_gather(rows_ref, [rowvs[c], colv])
            plsc.store_scatter(sbuf, [colv * 16 + lane], a)
        emb = [sbuf[pl.ds(ch * 16, 16)] for ch in range(28)]
        sdfs = emb[0]
        col0 = jnp.full((16,), 0, _i32)
        r0 = [plsc.load_gather(rows_ref, [rowvs[c], col0]) for c in range(8)]
        gx = jnp.zeros((16,), _f32)
        gy = jnp.zeros((16,), _f32)
        gz = jnp.zeros((16,), _f32)
        for j in range(4):
            gx = gx + wyz[j] * (r0[4 + j] - r0[j])
        for j in range(4):
            dxb, dzb = j >> 1, j & 1
            gy = gy + wxz[j] * (r0[dxb * 4 + 2 + dzb] - r0[dxb * 4 + dzb])
        for j in range(4):
            gz = gz + wxy[j] * (r0[2 * j + 1] - r0[2 * j])
        gx = gx * _f32(_INV_VOX)
        gy = gy * _f32(_INV_VOX)
        gz = gz * _f32(_INV_VOX)
        logits = [jnp.zeros((16,), _f32) for _ in range(3)]
        for k in range(3):
            for j in range(9):
                logits[k] = logits[k] + emb[1 + k * 9 + j] * comps[j]
        rgb = [1.0 / (1.0 + jnp.exp(-logits[k])) for k in range(3)]
        sgn = jnp.sign(sdfs)
        em1 = jnp.exp(-jnp.abs(sdfs) * inv_b) - 1.0
        sig = inv_b * (_f32(0.5) + _f32(0.5) * sgn * em1)
        sd_g = sig * dt
        s_r0[pl.ds(off, 16)] = rgb[0]
        s_r1[pl.ds(off, 16)] = rgb[1]
        s_r2[pl.ds(off, 16)] = rgb[2]
        s_tm[pl.ds(off, 16)] = tmid
        s_sd[pl.ds(off, 16)] = sd_g
        s_gx[pl.ds(off, 16)] = gx
        s_gy[pl.ds(off, 16)] = gy
        s_gz[pl.ds(off, 16)] = gz
        prev_g = plsc.load_gather(rid_v, [lane + (15 + off)])
        flag = rid_g != prev_g
        cin = plsc.cumsum(sd_g)
        excl = cin - sd_g
        total = jnp.max(cin)
        sv = jnp.where(flag, excl, _f32(_NEG))
        m = jnp.max(sv)
        hasg = jnp.max(flag.astype(_f32))
        acc = jnp.where(hasg > 0, total - m, acc + total)
        hasc = jnp.maximum(hasc, hasg)
        return (acc, hasc)

    def body(gg, carry):
        gs = [gg * 4 + i for i in range(4)]
        bufs = ((idx0, rows0, sem0), (idx1, rows1, sem1),
                (idx2, rows2, sem2), (idx3, rows3, sem3))
        pres, cps = [], []
        for g, (idx_r, rows_r, sem_r) in zip(gs, bufs):
            pres.append(pregather(g, idx_r))
            cps.append(pltpu.async_copy(table_hbm.at[idx_r], rows_r, sem_r))
        for g, (idx_r, rows_r, sem_r), pre, cp in zip(gs, bufs, pres, cps):
            cp.wait()
            carry = combine(g, rows_r, pre, carry)
        return carry

    acc, hasc = lax.fori_loop(0, _NGRP // 4, body, (_f32(0.0), _f32(0.0)))

    for r, ref in enumerate((s_r0, s_r1, s_r2, s_tm, s_sd, s_gx, s_gy, s_gz)):
        pltpu.sync_copy(ref, pert_hbm.at[r, pl.ds(base, _CHUNK)])
    tn_v[pl.ds(0, 16)] = jnp.broadcast_to(acc, (16,))
    pltpu.sync_copy(tn_v.at[pl.ds(0, 16)], tails_hbm.at[wid])
    tf_v[pl.ds(0, 16)] = jnp.broadcast_to(hasc, (16,))
    pltpu.sync_copy(tf_v.at[pl.ds(0, 16)], has_hbm.at[wid])


def _k3_body(pert_hbm, rid_hbm, tails_hbm, has_hbm, zeros_hbm,
             part_hbm, grads_hbm,
             s_r0, s_r1, s_r2, s_tm, s_sd, s_gx, s_gy, s_gz,
             rid_v, rid2d, tails_v, has_v, scr, vals_v, grads_v,
             acc_sh):
    cid = lax.axis_index("c")
    sid = lax.axis_index("s")
    wid = sid * _NC + cid
    base = wid * _CHUNK
    lane = lax.iota(_i32, 16)

    pltpu.sync_copy(zeros_hbm, vals_v.at[pl.ds(0, 256)])
    pltpu.sync_copy(vals_v.at[pl.ds(0, 256)], acc_sh.at[pl.ds(sid * 256, 256)])

    for r, ref in enumerate((s_r0, s_r1, s_r2, s_tm, s_sd, s_gx, s_gy, s_gz)):
        pltpu.sync_copy(pert_hbm.at[r, pl.ds(base, _CHUNK)], ref)

    @pl.when(wid == 0)
    def _():
        rid_v[pl.ds(0, 16)] = jnp.full((16,), -1, _i32)
        pltpu.sync_copy(rid_hbm.at[pl.ds(0, _CHUNK)], rid_v.at[pl.ds(16, _CHUNK)])

    @pl.when(wid > 0)
    def _():
        pltpu.sync_copy(rid_hbm.at[pl.ds(base - 16, _CHUNK + 16)], rid_v)

    for j in range(32):
        pltpu.sync_copy(rid_hbm.at[pl.ds(base + j * 128, 128)], rid2d.at[j])

    pltpu.sync_copy(tails_hbm, tails_v)
    pltpu.sync_copy(has_hbm, has_v)

    z16 = jnp.full((16,), 0, _i32)
    ta = plsc.load_gather(tails_v, [lane, z16])
    tb = plsc.load_gather(tails_v, [lane + 16, z16])
    ha = plsc.load_gather(has_v, [lane, z16])
    hb = plsc.load_gather(has_v, [lane + 16, z16])
    pia = plsc.cumsum(ta)
    pib = plsc.cumsum(tb) + jnp.max(pia)
    pea = pia - ta
    peb = pib - tb
    sva = jnp.where(ha > 0, pea, _f32(_NEG))
    svb = jnp.where(hb > 0, peb, _f32(_NEG))
    cma = plsc.cummax(sva)
    cmb = jnp.maximum(plsc.cummax(svb), jnp.max(cma))
    scr[pl.ds(0, 16)] = jnp.broadcast_to(_f32(0.0), (16,))
    scr[pl.ds(16, 16)] = cma
    scr[pl.ds(32, 16)] = cmb
    e2a = plsc.load_gather(scr, [lane + 15])
    e2b = plsc.load_gather(scr, [lane + 31])
    cva = pea - e2a
    cvb = peb - e2b
    sel_a = jnp.sum(jnp.where(lane == wid, cva, _f32(0.0)))
    sel_b = jnp.sum(jnp.where(lane == (wid - 16), cvb, _f32(0.0)))
    carry0 = jnp.where(wid < 16, sel_a, sel_b)

    plsc.subcore_barrier()

    def body(g, acc):
        off = g * 16
        sd_g = s_sd[pl.ds(off, 16)]
        rid_g = rid_v[pl.ds(16 + off, 16)]
        prev_g = plsc.load_gather(rid_v, [lane + (15 + off)])
        flag = rid_g != prev_g
        cin = plsc.cumsum(sd_g)
        excl = cin - sd_g
        total = jnp.max(cin)
        sv = jnp.where(flag, excl, _f32(_NEG))
        cmx = plsc.cummax(sv)
        e_lane = jnp.maximum(cmx, -acc)
        slocal = excl - e_lane
        trans = jnp.exp(-slocal)
        alpha = 1.0 - jnp.exp(-sd_g)
        wgt = alpha * trans
        m = jnp.max(sv)
        hasg = jnp.max(flag.astype(_f32))
        acc = jnp.where(hasg > 0, total - m, acc + total)
        gx = s_gx[pl.ds(off, 16)]
        gy = s_gy[pl.ds(off, 16)]
        gz = s_gz[pl.ds(off, 16)]
        gg = gx * gx + gy * gy + gz * gz
        bits = plsc.bitcast(gg, _i32)
        bits = jnp.full((16,), 0x5F3759DF, _i32) - lax.shift_right_logical(bits, 1)
        y = plsc.bitcast(bits, _f32)
        for _ in range(3):
            y = y * (_f32(1.5) - _f32(0.5) * gg * y * y)
        nlen = gg * y
        inv = 1.0 / jnp.maximum(nlen, _f32(1e-12))
        rowi = lane + off
        for k, gv in enumerate((gx, gy, gz)):
            plsc.store_scatter(grads_v, [rowi, jnp.full((16,), k, _i32)], gv)
        vals = (wgt * s_r0[pl.ds(off, 16)],
                wgt * s_r1[pl.ds(off, 16)],
                wgt * s_r2[pl.ds(off, 16)],
                wgt * s_tm[pl.ds(off, 16)],
                wgt * (gx * inv),
                wgt * (gy * inv),
                wgt * (gz * inv),
                wgt)
        for ch, v in enumerate(vals):
            plsc.store_scatter(vals_v, [rowi, jnp.full((16,), ch, _i32)], v)
        return acc

    lax.fori_loop(0, _NGRP, body, carry0)

    pltpu.sync_copy(grads_v, grads_hbm.at[pl.ds(base, _CHUNK)])

    def scat(j, carry):
        pltpu.sync_copy(vals_v.at[pl.ds(j * 128, 128)], acc_sh.at[rid2d.at[j]],
                        add=True)
        return carry

    lax.fori_loop(0, 32, scat, 0)

    plsc.subcore_barrier()

    @pl.when(sid == 0)
    def _():
        pltpu.sync_copy(acc_sh, part_hbm.at[cid])


def _k4_body(p_ref, rn_ref, rf_ref, rdn_ref,
             rgb_ref, depth_ref, nrm_ref, acc_ref, near_ref, far_ref):
    s = p_ref[0] + p_ref[1]
    rdn = rdn_ref[...]
    rgb_ref[...] = s[:, 0:3]
    depth_ref[...] = s[:, 3:4] / rdn
    nrm_ref[...] = s[:, 4:7]
    acc_ref[...] = s[:, 7:8]
    near_ref[...] = rn_ref[...] / rdn
    far_ref[...] = rf_ref[...] / rdn


_mesh = plsc.VectorSubcoreMesh(core_axis_name="c", subcore_axis_name="s")
_sc_params = pltpu.CompilerParams(needs_layout_passes=False,
                                  use_tc_tiling_on_sc=False)

_k1 = functools.partial(
    pl.kernel,
    out_type=(
        jax.ShapeDtypeStruct((8, _NSAMP), _f32),
        jax.ShapeDtypeStruct((_NW, 16), _f32),
        jax.ShapeDtypeStruct((_NW, 16), _f32),
    ),
    mesh=_mesh,
    compiler_params=_sc_params,
    scratch_types=[
        pltpu.VMEM((_NRAYS * 3,), _f32),
        pltpu.VMEM((_NRAYS * 3,), _f32),
        pltpu.VMEM((_CHUNK,), _f32),
        pltpu.VMEM((_CHUNK,), _f32),
        pltpu.VMEM((_CHUNK + 16,), _i32),
        pltpu.VMEM((16,), _f32),
        pltpu.VMEM((128,), _i32),
        pltpu.VMEM((128,), _i32),
        pltpu.VMEM((128,), _i32),
        pltpu.VMEM((128,), _i32),
        pltpu.VMEM((128, 32), _f32),
        pltpu.VMEM((128, 32), _f32),
        pltpu.VMEM((128, 32), _f32),
        pltpu.VMEM((128, 32), _f32),
        pltpu.VMEM((512,), _f32),
    ] + [pltpu.VMEM((_CHUNK,), _f32)] * 8
    + [pltpu.SemaphoreType.DMA] * 4,
)(_k1_body)

_k3 = functools.partial(
    pl.kernel,
    out_type=(
        jax.ShapeDtypeStruct((_NC, _NRAYS, 8), _f32),
        jax.ShapeDtypeStruct((_NSAMP, 3), _f32),
    ),
    mesh=_mesh,
    compiler_params=_sc_params,
    scratch_types=[pltpu.VMEM((_CHUNK,), _f32)] * 8
    + [
        pltpu.VMEM((_CHUNK + 16,), _i32),
        pltpu.VMEM((32, 128), _i32),
        pltpu.VMEM((_NW, 16), _f32),
        pltpu.VMEM((_NW, 16), _f32),
        pltpu.VMEM((48,), _f32),
        pltpu.VMEM((_CHUNK, 8), _f32),
        pltpu.VMEM((_CHUNK, 3), _f32),
        pltpu.VMEM_SHARED((_NRAYS, 8), _f32),
    ],
)(_k3_body)


def kernel(rays_o, rays_d, rays_d_norm, rays_near, rays_far, t_nears, t_fars,
           table, beta, ray_indices):
    tablep = jnp.pad(table, ((0, 0), (0, 4)))
    ro = rays_o.reshape(-1)
    rd = rays_d.reshape(-1)
    tn = t_nears.reshape(-1)
    tf = t_fars.reshape(-1)
    rid = ray_indices.astype(_i32)
    beta16 = jnp.broadcast_to(beta.reshape(1), (16,))
    zeros256 = jnp.zeros((256, 8), _f32)

    pert, tails, has = _k1(ro, rd, tn, tf, rid, tablep, beta16)
    partials, grads = _k3(pert, rid, tails, has, zeros256)

    rgb, depth, nrm, acc, near, far = pl.pallas_call(
        _k4_body,
        out_shape=(
            jax.ShapeDtypeStruct((_NRAYS, 3), _f32),
            jax.ShapeDtypeStruct((_NRAYS, 1), _f32),
            jax.ShapeDtypeStruct((_NRAYS, 3), _f32),
            jax.ShapeDtypeStruct((_NRAYS, 1), _f32),
            jax.ShapeDtypeStruct((_NRAYS, 1), _f32),
            jax.ShapeDtypeStruct((_NRAYS, 1), _f32),
        ),
    )(partials, rays_near, rays_far, rays_d_norm)

    return (rgb, depth, nrm, acc, grads, near, far)


# speedup vs baseline: 4.8512x; 1.0044x over previous
import functools

import jax
import jax.numpy as jnp
from jax import lax
from jax.experimental import pallas as pl
from jax.experimental.pallas import tpu as pltpu
from jax.experimental.pallas import tpu_sc as plsc

_VOX = 0.015
_INV_VOX = 1.0 / _VOX
_NEMB = 200000
_NRAYS = 4096
_NSAMP = 131072
_NC = 2
_NS = 16
_NW = _NC * _NS
_CHUNK = _NSAMP // _NW
_NGRP = _CHUNK // 16
_H1 = 2654435761
_H2 = 805459861
_NEG = -3.0e38

_f32 = jnp.float32
_i32 = jnp.int32


def _floor_frac(x):
    xi = x * _f32(_INV_VOX)
    t = xi.astype(_i32)
    tf = t.astype(_f32)
    b = t - (tf > xi).astype(_i32)
    f = xi - b.astype(_f32)
    return b, f


def _hash_parts(b, mult):
    u0 = plsc.bitcast(b, jnp.uint32) * jnp.uint32(mult)
    u1 = plsc.bitcast(b + 1, jnp.uint32) * jnp.uint32(mult)
    return u0, u1


def _sh_comps(dx, dy, dz):
    one = jnp.full((16,), 1.0, _f32)
    return [
        _f32(0.28209479177387814) * one,
        _f32(0.4886025119029199) * dy,
        _f32(0.4886025119029199) * dz,
        _f32(0.4886025119029199) * dx,
        _f32(1.0925484305920792) * dx * dy,
        _f32(1.0925484305920792) * dy * dz,
        _f32(0.9461746957575601) * dz * dz - _f32(0.31539156525252),
        _f32(1.0925484305920792) * dx * dz,
        _f32(0.5462742152960396) * (dx * dx - dy * dy),
    ]


def _k1_body(ro_hbm, rd_hbm, tn_hbm, tf_hbm, rid_hbm, table_hbm, beta_hbm,
             pert_hbm, tails_hbm, has_hbm,
             ro_v, rd_v, tn_v, tf_v, rid_v, beta_v,
             idx0, idx1, idx2, idx3, rows0, rows1, rows2, rows3, sbuf,
             s_r0, s_r1, s_r2, s_tm, s_sd, s_gx, s_gy, s_gz,
             sem0, sem1, sem2, sem3):
    cid = lax.axis_index("c")
    sid = lax.axis_index("s")
    wid = sid * _NC + cid
    base = wid * _CHUNK
    lane = lax.iota(_i32, 16)

    pltpu.sync_copy(ro_hbm, ro_v)
    pltpu.sync_copy(rd_hbm, rd_v)
    pltpu.sync_copy(beta_hbm, beta_v)
    pltpu.sync_copy(tn_hbm.at[pl.ds(base, _CHUNK)], tn_v)
    pltpu.sync_copy(tf_hbm.at[pl.ds(base, _CHUNK)], tf_v)

    @pl.when(wid == 0)
    def _():
        rid_v[pl.ds(0, 16)] = jnp.full((16,), -1, _i32)
        pltpu.sync_copy(rid_hbm.at[pl.ds(0, _CHUNK)], rid_v.at[pl.ds(16, _CHUNK)])

    @pl.when(wid > 0)
    def _():
        pltpu.sync_copy(rid_hbm.at[pl.ds(base - 16, _CHUNK + 16)], rid_v)

    bvec = _f32(_VOX) + jnp.abs(beta_v[...])
    inv_b = 1.0 / bvec

    def pregather(g, idx_ref):
        off = g * 16
        rid_g = rid_v[pl.ds(16 + off, 16)]
        tn_g = tn_v[pl.ds(off, 16)]
        tf_g = tf_v[pl.ds(off, 16)]
        tmid = _f32(0.5) * (tn_g + tf_g)
        dt = tf_g - tn_g
        b3 = rid_g * 3
        ox = plsc.load_gather(ro_v, [b3])
        oy = plsc.load_gather(ro_v, [b3 + 1])
        oz = plsc.load_gather(ro_v, [b3 + 2])
        ddx = plsc.load_gather(rd_v, [b3])
        ddy = plsc.load_gather(rd_v, [b3 + 1])
        ddz = plsc.load_gather(rd_v, [b3 + 2])
        b0, f0 = _floor_frac(ox + tmid * ddx)
        b1, f1 = _floor_frac(oy + tmid * ddy)
        b2, f2 = _floor_frac(oz + tmid * ddz)
        hx = _hash_parts(b0, 1)
        hy = _hash_parts(b1, _H1)
        hz = _hash_parts(b2, _H2)
        for c in range(8):
            dxb, dyb, dzb = (c >> 2) & 1, (c >> 1) & 1, c & 1
            h = (hx[dxb] ^ hy[dyb] ^ hz[dzb]) % jnp.uint32(_NEMB)
            idx_ref[pl.ds(c * 16, 16)] = h.astype(_i32)
        wx = [1.0 - f0, f0]
        wy = [1.0 - f1, f1]
        wz = [1.0 - f2, f2]
        wyz = [wy[j >> 1] * wz[j & 1] for j in range(4)]
        wxz = [wx[j >> 1] * wz[j & 1] for j in range(4)]
        wxy = [wx[j >> 1] * wy[j & 1] for j in range(4)]
        w8 = [wx[(c >> 2) & 1] * wyz[c & 3] for c in range(8)]
        comps = _sh_comps(ddx, ddy, ddz)
        return (rid_g, tmid, dt, w8, wyz, wxz, wxy, comps)

    def combine(g, rows_ref, pre, carry):
        acc, hasc = carry
        rid_g, tmid, dt, w8, wyz, wxz, wxy, comps = pre
        off = g * 16
        rowvs = [lane + (c * 16) for c in range(8)]
        for d in range(32):
            colv = (lane + d) & 31
            a = w8[0] * plsc.load_gather(rows_ref, [rowvs[0], colv])
            for c in range(1, 8):
                a = a + w8[c] * plsc.load_gather(rows_ref, [rowvs[c], colv])
            plsc.store_scatter(sbuf, [colv * 16 + lane], a)
        emb = [sbuf[pl.ds(ch * 16, 16)] for ch in range(28)]
        sdfs = emb[0]
        col0 = jnp.full((16,), 0, _i32)
        r0 = [plsc.load_gather(rows_ref, [rowvs[c], col0]) for c in range(8)]
        gx = jnp.zeros((16,), _f32)
        gy = jnp.zeros((16,), _f32)
        gz = jnp.zeros((16,), _f32)
        for j in range(4):
            gx = gx + wyz[j] * (r0[4 + j] - r0[j])
        for j in range(4):
            dxb, dzb = j >> 1, j & 1
            gy = gy + wxz[j] * (r0[dxb * 4 + 2 + dzb] - r0[dxb * 4 + dzb])
        for j in range(4):
            gz = gz + wxy[j] * (r0[2 * j + 1] - r0[2 * j])
        gx = gx * _f32(_INV_VOX)
        gy = gy * _f32(_INV_VOX)
        gz = gz * _f32(_INV_VOX)
        logits = [jnp.zeros((16,), _f32) for _ in range(3)]
        for k in range(3):
            for j in range(9):
                logits[k] = logits[k] + emb[1 + k * 9 + j] * comps[j]
        rgb = [1.0 / (1.0 + jnp.exp(-logits[k])) for k in range(3)]
        sgn = jnp.sign(sdfs)
        em1 = jnp.exp(-jnp.abs(sdfs) * inv_b) - 1.0
        sig = inv_b * (_f32(0.5) + _f32(0.5) * sgn * em1)
        sd_g = sig * dt
        s_r0[pl.ds(off, 16)] = rgb[0]
        s_r1[pl.ds(off, 16)] = rgb[1]
        s_r2[pl.ds(off, 16)] = rgb[2]
        s_tm[pl.ds(off, 16)] = tmid
        s_sd[pl.ds(off, 16)] = sd_g
        s_gx[pl.ds(off, 16)] = gx
        s_gy[pl.ds(off, 16)] = gy
        s_gz[pl.ds(off, 16)] = gz
        prev_g = plsc.load_gather(rid_v, [lane + (15 + off)])
        flag = rid_g != prev_g
        cin = plsc.cumsum(sd_g)
        excl = cin - sd_g
        total = jnp.max(cin)
        sv = jnp.where(flag, excl, _f32(_NEG))
        m = jnp.max(sv)
        hasg = jnp.max(flag.astype(_f32))
        acc = jnp.where(hasg > 0, total - m, acc + total)
        hasc = jnp.maximum(hasc, hasg)
        return (acc, hasc)

    def body(gg, carry):
        gs = [gg * 4 + i for i in range(4)]
        bufs = ((idx0, rows0, sem0), (idx1, rows1, sem1),
                (idx2, rows2, sem2), (idx3, rows3, sem3))
        pres, cps = [], []
        for g, (idx_r, rows_r, sem_r) in zip(gs, bufs):
            pres.append(pregather(g, idx_r))
            cps.append(pltpu.async_copy(table_hbm.at[idx_r], rows_r, sem_r))
        for g, (idx_r, rows_r, sem_r), pre, cp in zip(gs, bufs, pres, cps):
            cp.wait()
            carry = combine(g, rows_r, pre, carry)
        return carry

    acc, hasc = lax.fori_loop(0, _NGRP // 4, body, (_f32(0.0), _f32(0.0)))

    for r, ref in enumerate((s_r0, s_r1, s_r2, s_tm, s_sd, s_gx, s_gy, s_gz)):
        pltpu.sync_copy(ref, pert_hbm.at[r, pl.ds(base, _CHUNK)])
    tn_v[pl.ds(0, 16)] = jnp.broadcast_to(acc, (16,))
    pltpu.sync_copy(tn_v.at[pl.ds(0, 16)], tails_hbm.at[wid])
    tf_v[pl.ds(0, 16)] = jnp.broadcast_to(hasc, (16,))
    pltpu.sync_copy(tf_v.at[pl.ds(0, 16)], has_hbm.at[wid])


def _k3_body(pert_hbm, rid_hbm, tails_hbm, has_hbm, zeros_hbm,
             part_hbm, grads_hbm,
             s_r0, s_r1, s_r2, s_tm, s_sd, s_gx, s_gy, s_gz,
             rid_v, rid2d, tails_v, has_v, scr, vals_v, grads_v,
             acc_sh):
    cid = lax.axis_index("c")
    sid = lax.axis_index("s")
    wid = sid * _NC + cid
    base = wid * _CHUNK
    lane = lax.iota(_i32, 16)

    pltpu.sync_copy(zeros_hbm, vals_v.at[pl.ds(0, 256)])
    pltpu.sync_copy(vals_v.at[pl.ds(0, 256)], acc_sh.at[pl.ds(sid * 256, 256)])

    for r, ref in enumerate((s_r0, s_r1, s_r2, s_tm, s_sd, s_gx, s_gy, s_gz)):
        pltpu.sync_copy(pert_hbm.at[r, pl.ds(base, _CHUNK)], ref)

    @pl.when(wid == 0)
    def _():
        rid_v[pl.ds(0, 16)] = jnp.full((16,), -1, _i32)
        pltpu.sync_copy(rid_hbm.at[pl.ds(0, _CHUNK)], rid_v.at[pl.ds(16, _CHUNK)])

    @pl.when(wid > 0)
    def _():
        pltpu.sync_copy(rid_hbm.at[pl.ds(base - 16, _CHUNK + 16)], rid_v)

    for j in range(32):
        pltpu.sync_copy(rid_hbm.at[pl.ds(base + j * 128, 128)], rid2d.at[j])

    pltpu.sync_copy(tails_hbm, tails_v)
    pltpu.sync_copy(has_hbm, has_v)

    z16 = jnp.full((16,), 0, _i32)
    ta = plsc.load_gather(tails_v, [lane, z16])
    tb = plsc.load_gather(tails_v, [lane + 16, z16])
    ha = plsc.load_gather(has_v, [lane, z16])
    hb = plsc.load_gather(has_v, [lane + 16, z16])
    pia = plsc.cumsum(ta)
    pib = plsc.cumsum(tb) + jnp.max(pia)
    pea = pia - ta
    peb = pib - tb
    sva = jnp.where(ha > 0, pea, _f32(_NEG))
    svb = jnp.where(hb > 0, peb, _f32(_NEG))
    cma = plsc.cummax(sva)
    cmb = jnp.maximum(plsc.cummax(svb), jnp.max(cma))
    scr[pl.ds(0, 16)] = jnp.broadcast_to(_f32(0.0), (16,))
    scr[pl.ds(16, 16)] = cma
    scr[pl.ds(32, 16)] = cmb
    e2a = plsc.load_gather(scr, [lane + 15])
    e2b = plsc.load_gather(scr, [lane + 31])
    cva = pea - e2a
    cvb = peb - e2b
    sel_a = jnp.sum(jnp.where(lane == wid, cva, _f32(0.0)))
    sel_b = jnp.sum(jnp.where(lane == (wid - 16), cvb, _f32(0.0)))
    carry0 = jnp.where(wid < 16, sel_a, sel_b)

    plsc.subcore_barrier()

    def body(g, acc):
        off = g * 16
        sd_g = s_sd[pl.ds(off, 16)]
        rid_g = rid_v[pl.ds(16 + off, 16)]
        prev_g = plsc.load_gather(rid_v, [lane + (15 + off)])
        flag = rid_g != prev_g
        cin = plsc.cumsum(sd_g)
        excl = cin - sd_g
        total = jnp.max(cin)
        sv = jnp.where(flag, excl, _f32(_NEG))
        cmx = plsc.cummax(sv)
        e_lane = jnp.maximum(cmx, -acc)
        slocal = excl - e_lane
        trans = jnp.exp(-slocal)
        alpha = 1.0 - jnp.exp(-sd_g)
        wgt = alpha * trans
        m = jnp.max(sv)
        hasg = jnp.max(flag.astype(_f32))
        acc = jnp.where(hasg > 0, total - m, acc + total)
        gx = s_gx[pl.ds(off, 16)]
        gy = s_gy[pl.ds(off, 16)]
        gz = s_gz[pl.ds(off, 16)]
        gg = gx * gx + gy * gy + gz * gz
        bits = plsc.bitcast(gg, _i32)
        bits = jnp.full((16,), 0x5F3759DF, _i32) - lax.shift_right_logical(bits, 1)
        y = plsc.bitcast(bits, _f32)
        for _ in range(3):
            y = y * (_f32(1.5) - _f32(0.5) * gg * y * y)
        nlen = gg * y
        inv = 1.0 / jnp.maximum(nlen, _f32(1e-12))
        rowi = lane + off
        for k, gv in enumerate((gx, gy, gz)):
            plsc.store_scatter(grads_v, [rowi, jnp.full((16,), k, _i32)], gv)
        vals = (wgt * s_r0[pl.ds(off, 16)],
                wgt * s_r1[pl.ds(off, 16)],
                wgt * s_r2[pl.ds(off, 16)],
                wgt * s_tm[pl.ds(off, 16)],
                wgt * (gx * inv),
                wgt * (gy * inv),
                wgt * (gz * inv),
                wgt)
        for ch, v in enumerate(vals):
            plsc.store_scatter(vals_v, [rowi, jnp.full((16,), ch, _i32)], v)
        return acc

    lax.fori_loop(0, _NGRP, body, carry0)

    pltpu.sync_copy(grads_v, grads_hbm.at[pl.ds(base, _CHUNK)])

    def scat(j, carry):
        pltpu.sync_copy(vals_v.at[pl.ds(j * 128, 128)], acc_sh.at[rid2d.at[j]],
                        add=True)
        return carry

    lax.fori_loop(0, 32, scat, 0)

    plsc.subcore_barrier()

    @pl.when(sid == 0)
    def _():
        pltpu.sync_copy(acc_sh, part_hbm.at[cid])


def _k4_body(p_ref, rn_ref, rf_ref, rdn_ref,
             rgb_ref, depth_ref, nrm_ref, acc_ref, near_ref, far_ref):
    s = p_ref[0] + p_ref[1]
    rdn = rdn_ref[...]
    rgb_ref[...] = s[:, 0:3]
    depth_ref[...] = s[:, 3:4] / rdn
    nrm_ref[...] = s[:, 4:7]
    acc_ref[...] = s[:, 7:8]
    near_ref[...] = rn_ref[...] / rdn
    far_ref[...] = rf_ref[...] / rdn


_mesh = plsc.VectorSubcoreMesh(core_axis_name="c", subcore_axis_name="s")
_sc_params = pltpu.CompilerParams(needs_layout_passes=False,
                                  use_tc_tiling_on_sc=False)

_k1 = functools.partial(
    pl.kernel,
    out_type=(
        jax.ShapeDtypeStruct((8, _NSAMP), _f32),
        jax.ShapeDtypeStruct((_NW, 16), _f32),
        jax.ShapeDtypeStruct((_NW, 16), _f32),
    ),
    mesh=_mesh,
    compiler_params=_sc_params,
    scratch_types=[
        pltpu.VMEM((_NRAYS * 3,), _f32),
        pltpu.VMEM((_NRAYS * 3,), _f32),
        pltpu.VMEM((_CHUNK,), _f32),
        pltpu.VMEM((_CHUNK,), _f32),
        pltpu.VMEM((_CHUNK + 16,), _i32),
        pltpu.VMEM((16,), _f32),
    ] + [pltpu.VMEM((128,), _i32)] * 4
    + [pltpu.VMEM((128, 32), _f32)] * 4
    + [
        pltpu.VMEM((512,), _f32),
    ] + [pltpu.VMEM((_CHUNK,), _f32)] * 8
    + [pltpu.SemaphoreType.DMA] * 4,
)(_k1_body)

_k3 = functools.partial(
    pl.kernel,
    out_type=(
        jax.ShapeDtypeStruct((_NC, _NRAYS, 8), _f32),
        jax.ShapeDtypeStruct((_NSAMP, 3), _f32),
    ),
    mesh=_mesh,
    compiler_params=_sc_params,
    scratch_types=[pltpu.VMEM((_CHUNK,), _f32)] * 8
    + [
        pltpu.VMEM((_CHUNK + 16,), _i32),
        pltpu.VMEM((32, 128), _i32),
        pltpu.VMEM((_NW, 16), _f32),
        pltpu.VMEM((_NW, 16), _f32),
        pltpu.VMEM((48,), _f32),
        pltpu.VMEM((_CHUNK, 8), _f32),
        pltpu.VMEM((_CHUNK, 3), _f32),
        pltpu.VMEM_SHARED((_NRAYS, 8), _f32),
    ],
)(_k3_body)


def kernel(rays_o, rays_d, rays_d_norm, rays_near, rays_far, t_nears, t_fars,
           table, beta, ray_indices):
    tablep = jnp.pad(table, ((0, 0), (0, 4)))
    ro = rays_o.reshape(-1)
    rd = rays_d.reshape(-1)
    tn = t_nears.reshape(-1)
    tf = t_fars.reshape(-1)
    rid = ray_indices.astype(_i32)
    beta16 = jnp.broadcast_to(beta.reshape(1), (16,))
    zeros256 = jnp.zeros((256, 8), _f32)

    pert, tails, has = _k1(ro, rd, tn, tf, rid, tablep, beta16)
    partials, grads = _k3(pert, rid, tails, has, zeros256)

    rgb, depth, nrm, acc, near, far = pl.pallas_call(
        _k4_body,
        out_shape=(
            jax.ShapeDtypeStruct((_NRAYS, 3), _f32),
            jax.ShapeDtypeStruct((_NRAYS, 1), _f32),
            jax.ShapeDtypeStruct((_NRAYS, 3), _f32),
            jax.ShapeDtypeStruct((_NRAYS, 1), _f32),
            jax.ShapeDtypeStruct((_NRAYS, 1), _f32),
            jax.ShapeDtypeStruct((_NRAYS, 1), _f32),
        ),
    )(partials, rays_near, rays_far, rays_d_norm)

    return (rgb, depth, nrm, acc, grads, near, far)
